# baseline (speedup 1.0000x reference)
import functools

import jax
import jax.numpy as jnp
from jax import lax
from jax.experimental import pallas as pl
from jax.experimental.pallas import tpu as pltpu

_N_ANCHORS = 20000
_PRE = 6000
_NPAD = 6144
_BLK = 512
_NBLK = _NPAD // _BLK
_PROP = 1000
_PROP_PAD = 1024
_THR = 0.7
_STD = (0.1, 0.1, 0.2, 0.2)


def _iota1d(n, dtype=jnp.int32):
    return lax.broadcasted_iota(dtype, (1, n), 1)[0]


def _iou_tile(xr1, yr1, xr2, yr2, ar, xc1, yc1, xc2, yc2, ac):
    ix1 = jnp.maximum(xr1[:, None], xc1[None, :])
    iy1 = jnp.maximum(yr1[:, None], yc1[None, :])
    ix2 = jnp.minimum(xr2[:, None], xc2[None, :])
    iy2 = jnp.minimum(yr2[:, None], yc2[None, :])
    inter = jnp.maximum(ix2 - ix1, 0.0) * jnp.maximum(iy2 - iy1, 0.0)
    union = ar[:, None] + ac[None, :] - inter
    return inter / jnp.maximum(union, 1e-12)


def _nms_kernel(a_ref, d_ref, out_ref, st_ref):
    f32 = jnp.float32

    ax1 = a_ref[0, 0, :]
    ay1 = a_ref[0, 1, :]
    ax2 = a_ref[0, 2, :]
    ay2 = a_ref[0, 3, :]
    d0 = d_ref[0, 0, :]
    d1 = d_ref[0, 1, :]
    e2 = d_ref[0, 2, :]
    e3 = d_ref[0, 3, :]

    w = ax2 - ax1
    h = ay2 - ay1
    cx = ax1 + 0.5 * w
    cy = ay1 + 0.5 * h
    cx = cx + d0 * w
    cy = cy + d1 * h
    w = w * e2
    h = h * e3
    nx1 = cx - 0.5 * w
    ny1 = cy - 0.5 * h
    nx2 = nx1 + w
    ny2 = ny1 + h
    x1 = jnp.clip(nx1, 0.0, 1.0)
    y1 = jnp.clip(ny1, 0.0, 1.0)
    x2 = jnp.clip(nx2, 0.0, 1.0)
    y2 = jnp.clip(ny2, 0.0, 1.0)
    area = jnp.maximum(x2 - x1, 0.0) * jnp.maximum(y2 - y1, 0.0)

    st_ref[0, :] = x1
    st_ref[1, :] = y1
    st_ref[2, :] = x2
    st_ref[3, :] = y2
    st_ref[4, :] = area
    st_ref[5, :] = jnp.zeros((_NPAD,), f32)

    rix = lax.broadcasted_iota(jnp.int32, (_BLK, _BLK), 0)
    cix = lax.broadcasted_iota(jnp.int32, (_BLK, _BLK), 1)
    upper = (rix < cix).astype(f32)

    def blk_body(k, _):
        s0 = pl.multiple_of(k * _BLK, _BLK)
        ds = pl.ds(s0, _BLK)
        xr1 = st_ref[0, ds]
        yr1 = st_ref[1, ds]
        xr2 = st_ref[2, ds]
        yr2 = st_ref[3, ds]
        ar = st_ref[4, ds]
        sup = st_ref[5, ds]
        gidx = s0 + _iota1d(_BLK)
        valid = (gidx < _PRE).astype(f32)
        alive = ((1.0 - sup) * valid).reshape(1, _BLK)

        iou_kk = _iou_tile(xr1, yr1, xr2, yr2, ar, xr1, yr1, xr2, yr2, ar)
        ms = (iou_kk > _THR).astype(f32) * upper

        def w_cond(c):
            _, changed, it = c
            return jnp.logical_and(changed, it < _BLK + 2)

        def w_body(c):
            s, _, it = c
            hit = jnp.dot(s, ms, preferred_element_type=f32)
            s2 = alive * (hit <= 0.0).astype(f32)
            return (s2, jnp.any(s2 != s), it + 1)

        s_fin, _, _ = lax.while_loop(
            w_cond, w_body, (alive, jnp.bool_(True), jnp.int32(0))
        )
        st_ref[6, ds] = s_fin[0]

        def j_body(j, _):
            c0 = pl.multiple_of(j * _BLK, _BLK)
            dc = pl.ds(c0, _BLK)
            iou_kj = _iou_tile(
                xr1, yr1, xr2, yr2, ar,
                st_ref[0, dc], st_ref[1, dc], st_ref[2, dc], st_ref[3, dc],
                st_ref[4, dc],
            )
            m = (iou_kj > _THR).astype(f32)
            hit = jnp.dot(s_fin, m, preferred_element_type=f32)
            newsup = (hit[0] > 0.0).astype(f32)
            st_ref[5, dc] = jnp.maximum(st_ref[5, dc], newsup)
            return 0

        lax.fori_loop(k + 1, _NBLK, j_body, 0)
        return 0

    lax.fori_loop(0, _NBLK, blk_body, 0)

    tri = (rix <= cix).astype(f32)

    def cs_body(k, off):
        ds = pl.ds(pl.multiple_of(k * _BLK, _BLK), _BLK)
        s = st_ref[6, ds].reshape(1, _BLK)
        c = jnp.dot(s, tri, preferred_element_type=f32) + off
        st_ref[7, ds] = c[0]
        return off + jnp.sum(s)

    lax.fori_loop(0, _NBLK, cs_body, jnp.float32(0.0))

    jio = lax.broadcasted_iota(jnp.int32, (_PROP_PAD, _BLK), 0).astype(f32) + 1.0

    def acc_body(k, acc):
        ds = pl.ds(pl.multiple_of(k * _BLK, _BLK), _BLK)
        cb = st_ref[7, ds].reshape(1, _BLK)
        sb = st_ref[6, ds].reshape(1, _BLK)
        e = (jio == cb).astype(f32) * sb
        boxes = jnp.concatenate(
            [st_ref[0, ds][:, None], st_ref[1, ds][:, None],
             st_ref[2, ds][:, None], st_ref[3, ds][:, None]], axis=1)
        return acc + jnp.dot(e, boxes, preferred_element_type=f32,
                             precision=lax.Precision.HIGHEST)

    acc = lax.fori_loop(
        0, _NBLK, acc_body, jnp.zeros((_PROP_PAD, 4), f32)
    )
    out_ref[0] = acc[:_PROP, :]


@functools.partial(jax.jit, static_argnames=("interpret",))
def _run(scores, deltas, anchors, interpret=False):
    fg = scores[:, :, 1]
    _, top_idx = lax.top_k(fg, _PRE)
    dd = deltas * jnp.asarray(_STD, jnp.float32).reshape(1, 1, 4)
    d = jnp.take_along_axis(dd, top_idx[:, :, None], axis=1)
    d = jnp.concatenate([d[:, :, :2], jnp.exp(d[:, :, 2:])], axis=-1)
    a = jnp.take_along_axis(anchors, top_idx[:, :, None], axis=1)
    pad = ((0, 0), (0, _NPAD - _PRE), (0, 0))
    d = jnp.pad(d, pad)
    a = jnp.pad(a, pad)
    d_t = jnp.transpose(d, (0, 2, 1))
    a_t = jnp.transpose(a, (0, 2, 1))

    batch = scores.shape[0]
    out = pl.pallas_call(
        _nms_kernel,
        grid=(batch,),
        in_specs=[
            pl.BlockSpec((1, 4, _NPAD), lambda b: (b, 0, 0)),
            pl.BlockSpec((1, 4, _NPAD), lambda b: (b, 0, 0)),
        ],
        out_specs=pl.BlockSpec((1, _PROP, 4), lambda b: (b, 0, 0)),
        out_shape=jax.ShapeDtypeStruct((batch, _PROP, 4), jnp.float32),
        scratch_shapes=[pltpu.VMEM((8, _NPAD), jnp.float32)],
        interpret=interpret,
    )(a_t, d_t)
    return out


def kernel(scores, deltas, anchors):
    return _run(scores, deltas, anchors)


# speedup vs baseline: 1.5363x; 1.5363x over previous
import functools

import jax
import jax.numpy as jnp
from jax import lax
from jax.experimental import pallas as pl
from jax.experimental.pallas import tpu as pltpu

_N_ANCHORS = 20000
_PRE = 6000
_NPAD = 6144
_BLK = 512
_NBLK = _NPAD // _BLK
_PROP = 1000
_PROP_PAD = 1024
_THR = 0.7
_STD = (0.1, 0.1, 0.2, 0.2)


def _iota1d(n, dtype=jnp.int32):
    return lax.broadcasted_iota(dtype, (1, n), 1)[0]


def _iou_tile(xr1, yr1, xr2, yr2, ar, xc1, yc1, xc2, yc2, ac):
    ix1 = jnp.maximum(xr1[:, None], xc1[None, :])
    iy1 = jnp.maximum(yr1[:, None], yc1[None, :])
    ix2 = jnp.minimum(xr2[:, None], xc2[None, :])
    iy2 = jnp.minimum(yr2[:, None], yc2[None, :])
    inter = jnp.maximum(ix2 - ix1, 0.0) * jnp.maximum(iy2 - iy1, 0.0)
    union = ar[:, None] + ac[None, :] - inter
    return inter / jnp.maximum(union, 1e-12)


def _nms_kernel(a_ref, d_ref, out_ref, st_ref):
    f32 = jnp.float32

    ax1 = a_ref[0, 0, :]
    ay1 = a_ref[0, 1, :]
    ax2 = a_ref[0, 2, :]
    ay2 = a_ref[0, 3, :]
    d0 = d_ref[0, 0, :]
    d1 = d_ref[0, 1, :]
    e2 = d_ref[0, 2, :]
    e3 = d_ref[0, 3, :]

    w = ax2 - ax1
    h = ay2 - ay1
    cx = ax1 + 0.5 * w
    cy = ay1 + 0.5 * h
    cx = cx + d0 * w
    cy = cy + d1 * h
    w = w * e2
    h = h * e3
    nx1 = cx - 0.5 * w
    ny1 = cy - 0.5 * h
    nx2 = nx1 + w
    ny2 = ny1 + h
    x1 = jnp.clip(nx1, 0.0, 1.0)
    y1 = jnp.clip(ny1, 0.0, 1.0)
    x2 = jnp.clip(nx2, 0.0, 1.0)
    y2 = jnp.clip(ny2, 0.0, 1.0)
    area = jnp.maximum(x2 - x1, 0.0) * jnp.maximum(y2 - y1, 0.0)

    st_ref[0, :] = x1
    st_ref[1, :] = y1
    st_ref[2, :] = x2
    st_ref[3, :] = y2
    st_ref[4, :] = area
    st_ref[6, :] = jnp.zeros((_NPAD,), f32)

    rix = lax.broadcasted_iota(jnp.int32, (_BLK, _BLK), 0)
    cix = lax.broadcasted_iota(jnp.int32, (_BLK, _BLK), 1)
    upper = (rix < cix).astype(f32)

    def blk_body(k, _):
        done = jnp.sum(st_ref[6, :]) >= jnp.float32(_PROP)

        @pl.when(jnp.logical_not(done))
        def _process():
            s0 = pl.multiple_of(k * _BLK, _BLK)
            ds = pl.ds(s0, _BLK)
            xc1 = st_ref[0, ds]
            yc1 = st_ref[1, ds]
            xc2 = st_ref[2, ds]
            yc2 = st_ref[3, ds]
            ac = st_ref[4, ds]
            gidx = s0 + _iota1d(_BLK)
            valid = (gidx < _PRE).astype(f32)

            def pull(j, supc):
                dc = pl.ds(pl.multiple_of(j * _BLK, _BLK), _BLK)
                iou_jk = _iou_tile(
                    st_ref[0, dc], st_ref[1, dc], st_ref[2, dc],
                    st_ref[3, dc], st_ref[4, dc],
                    xc1, yc1, xc2, yc2, ac,
                )
                m = (iou_jk > _THR).astype(f32)
                s_j = st_ref[6, dc].reshape(1, _BLK)
                hit = jnp.dot(s_j, m, preferred_element_type=f32)
                return jnp.maximum(supc, (hit > 0.0).astype(f32))

            sup = lax.fori_loop(0, k, pull, jnp.zeros((1, _BLK), f32))
            alive = (1.0 - sup) * valid.reshape(1, _BLK)

            iou_kk = _iou_tile(xc1, yc1, xc2, yc2, ac,
                               xc1, yc1, xc2, yc2, ac)
            ms = (iou_kk > _THR).astype(f32) * upper

            def w_cond(c):
                _, changed, it = c
                return jnp.logical_and(changed, it < _BLK + 2)

            def w_body(c):
                s, _, it = c
                hit = jnp.dot(s, ms, preferred_element_type=f32)
                s2 = alive * (hit <= 0.0).astype(f32)
                return (s2, jnp.any(s2 != s), it + 1)

            s_fin, _, _ = lax.while_loop(
                w_cond, w_body, (alive, jnp.bool_(True), jnp.int32(0))
            )
            st_ref[6, ds] = s_fin[0]

        return 0

    lax.fori_loop(0, _NBLK, blk_body, 0)

    tri = (rix <= cix).astype(f32)

    def cs_body(k, off):
        ds = pl.ds(pl.multiple_of(k * _BLK, _BLK), _BLK)
        s = st_ref[6, ds].reshape(1, _BLK)
        c = jnp.dot(s, tri, preferred_element_type=f32) + off
        st_ref[7, ds] = c[0]
        return off + jnp.sum(s)

    lax.fori_loop(0, _NBLK, cs_body, jnp.float32(0.0))

    jio = lax.broadcasted_iota(jnp.int32, (_PROP_PAD, _BLK), 0).astype(f32) + 1.0

    def acc_body(k, carry):
        acc, off = carry
        ds = pl.ds(pl.multiple_of(k * _BLK, _BLK), _BLK)
        sb = st_ref[6, ds].reshape(1, _BLK)

        def live(_):
            cb = st_ref[7, ds].reshape(1, _BLK)
            e = (jio == cb).astype(f32) * sb
            boxes = jnp.concatenate(
                [st_ref[0, ds][:, None], st_ref[1, ds][:, None],
                 st_ref[2, ds][:, None], st_ref[3, ds][:, None]], axis=1)
            return acc + jnp.dot(e, boxes, preferred_element_type=f32,
                                 precision=lax.Precision.HIGHEST)

        acc2 = lax.cond(off < jnp.float32(_PROP), live, lambda _: acc, 0)
        return (acc2, off + jnp.sum(sb))

    acc, _ = lax.fori_loop(
        0, _NBLK, acc_body, (jnp.zeros((_PROP_PAD, 4), f32), jnp.float32(0.0))
    )
    out_ref[0] = acc[:_PROP, :]


@functools.partial(jax.jit, static_argnames=("interpret",))
def _run(scores, deltas, anchors, interpret=False):
    fg = scores[:, :, 1]
    _, top_idx = lax.top_k(fg, _PRE)
    dd = deltas * jnp.asarray(_STD, jnp.float32).reshape(1, 1, 4)
    d = jnp.take_along_axis(dd, top_idx[:, :, None], axis=1)
    d = jnp.concatenate([d[:, :, :2], jnp.exp(d[:, :, 2:])], axis=-1)
    a = jnp.take_along_axis(anchors, top_idx[:, :, None], axis=1)
    pad = ((0, 0), (0, _NPAD - _PRE), (0, 0))
    d = jnp.pad(d, pad)
    a = jnp.pad(a, pad)
    d_t = jnp.transpose(d, (0, 2, 1))
    a_t = jnp.transpose(a, (0, 2, 1))

    batch = scores.shape[0]
    out = pl.pallas_call(
        _nms_kernel,
        grid=(batch,),
        in_specs=[
            pl.BlockSpec((1, 4, _NPAD), lambda b: (b, 0, 0)),
            pl.BlockSpec((1, 4, _NPAD), lambda b: (b, 0, 0)),
        ],
        out_specs=pl.BlockSpec((1, _PROP, 4), lambda b: (b, 0, 0)),
        out_shape=jax.ShapeDtypeStruct((batch, _PROP, 4), jnp.float32),
        scratch_shapes=[pltpu.VMEM((8, _NPAD), jnp.float32)],
        interpret=interpret,
    )(a_t, d_t)
    return out


def kernel(scores, deltas, anchors):
    return _run(scores, deltas, anchors)
